# baseline (speedup 1.0000x reference)
import functools

import jax
import jax.numpy as jnp
from jax import lax
from jax.experimental import pallas as pl
from jax.experimental.pallas import tpu as pltpu
from jax.experimental.pallas import tpu_sc as plsc

D_MODEL = 64
SCALE = 8.0
LANES = 16
NUM_CORES = 2
NUM_SUBCORES = 16
NUM_WORKERS = NUM_CORES * NUM_SUBCORES
CHUNK = 128
NBUF = 4
SUBLANES = 8
NGROUP = CHUNK // LANES
KF_NBUF = 3

_COMPILER_PARAMS = pltpu.CompilerParams(
    use_tc_tiling_on_sc=True,
    needs_layout_passes=False,
    disable_bounds_checks=True,
)


@functools.lru_cache(maxsize=None)
def _make_format_kernel(vocab: int):
    n_tiles = (vocab + CHUNK - 1) // CHUNK
    pairs = vocab // 2
    base_cnt = n_tiles // NUM_WORKERS
    rem = n_tiles % NUM_WORKERS
    max_cnt = base_cnt + (1 if rem else 0)
    n_rounds = (max_cnt + KF_NBUF - 1) // KF_NBUF
    last_tile = n_tiles - 1
    last_rows = (pairs - last_tile * (CHUNK // 2)) if vocab % CHUNK else 0
    mesh = plsc.VectorSubcoreMesh(core_axis_name="c", subcore_axis_name="s")

    @functools.partial(
        pl.kernel,
        out_type=jax.ShapeDtypeStruct((pairs, 2 * D_MODEL), jnp.float32),
        mesh=mesh,
        scratch_types=[pltpu.VMEM((D_MODEL, CHUNK), jnp.float32)
                       for _ in range(KF_NBUF)]
        + [pltpu.VMEM((D_MODEL, CHUNK), jnp.float32)
           for _ in range(KF_NBUF)]
        + [pltpu.SemaphoreType.DMA] * (2 * KF_NBUF),
        compiler_params=_COMPILER_PARAMS,
    )
    def fmt_kernel(tt_hbm, out_hbm, *rest):
        src = rest[:KF_NBUF]
        dst = rest[KF_NBUF : 2 * KF_NBUF]
        rsem = rest[2 * KF_NBUF : 3 * KF_NBUF]
        wsem = rest[3 * KF_NBUF :]
        wid = lax.axis_index("s") * NUM_CORES + lax.axis_index("c")
        start = wid * base_cnt + lax.min(wid, rem)
        count = base_cnt + jnp.where(wid < rem, 1, 0)

        iota = lax.iota(jnp.int32, LANES)
        d_vecs = [iota + jj * LANES for jj in range(D_MODEL // LANES)]

        def fire_read(t, k):
            vh = start + t
            pltpu.async_copy(
                tt_hbm.at[:, pl.ds(vh * CHUNK, CHUNK)], src[k], rsem[k]
            )

        def wait_read(t, k):
            vh = start + t
            pltpu.make_async_copy(
                tt_hbm.at[:, pl.ds(vh * CHUNK, CHUNK)], src[k], rsem[k]
            ).wait()

        def fire_write(t, k):
            vh = start + t
            row0 = vh * (CHUNK // 2)

            @pl.when(vh != last_tile)
            def _():
                pltpu.async_copy(
                    dst[k], out_hbm.at[pl.ds(row0, D_MODEL)], wsem[k]
                )

            if last_rows:
                @pl.when(vh == last_tile)
                def _():
                    pltpu.async_copy(
                        dst[k].at[pl.ds(0, last_rows)],
                        out_hbm.at[pl.ds(row0, last_rows)],
                        wsem[k],
                    )

        def wait_write(t, k):
            vh = start + t

            @pl.when(vh != last_tile)
            def _():
                pltpu.make_async_copy(
                    dst[k], out_hbm.at[pl.ds(0, D_MODEL)], wsem[k]
                ).wait()

            if last_rows:
                @pl.when(vh == last_tile)
                def _():
                    pltpu.make_async_copy(
                        dst[k].at[pl.ds(0, last_rows)],
                        out_hbm.at[pl.ds(0, last_rows)],
                        wsem[k],
                    ).wait()

        def transpose(k):
            def vl_body(vl, carry):
                col = jnp.full((LANES,), vl, jnp.int32)
                row = lax.shift_right_logical(vl, 1)
                cb = lax.shift_left(lax.bitwise_and(vl, 1), 6)
                for jj in range(D_MODEL // LANES):
                    v = plsc.load_gather(src[k], [d_vecs[jj], col])
                    dst[k][row, pl.ds(cb + jj * LANES, LANES)] = v * SCALE
                return carry

            lax.fori_loop(0, CHUNK, vl_body, 0, unroll=2)

        def round_body(r, carry):
            for k in range(KF_NBUF):
                t = r * KF_NBUF + k

                @pl.when(t < count)
                def _():
                    wait_read(t, k)

                    @pl.when(r > 0)
                    def _():
                        wait_write(t - KF_NBUF, k)

                    transpose(k)
                    fire_write(t, k)

                    @pl.when(t + KF_NBUF < count)
                    def _():
                        fire_read(t + KF_NBUF, k)

            return carry

        for k in range(KF_NBUF):
            fire_read(k, k)
        lax.fori_loop(0, n_rounds, round_body, 0)
        for k in range(KF_NBUF):
            last_t = count - 1 - lax.rem(count - 1 - k, jnp.int32(KF_NBUF))
            wait_write(last_t, k)

    return fmt_kernel


@functools.lru_cache(maxsize=None)
def _make_gather_kernel(batch: int, seq: int):
    n_rows = batch * seq
    assert batch % CHUNK == 0
    i_blocks = batch // CHUNK
    n_blocks = n_rows // CHUNK
    assert n_blocks % (NUM_WORKERS * NBUF) == 0
    n_chunks = n_blocks // NUM_WORKERS
    n_rounds = n_chunks // NBUF
    d_blocks = D_MODEL // SUBLANES
    mesh = plsc.VectorSubcoreMesh(core_axis_name="c", subcore_axis_name="s")

    @functools.partial(
        pl.kernel,
        out_type=jax.ShapeDtypeStruct(
            (seq, d_blocks, i_blocks, SUBLANES, CHUNK), jnp.float32
        ),
        mesh=mesh,
        scratch_types=[
            pltpu.VMEM((NBUF, CHUNK), jnp.int32),
            pltpu.VMEM((NBUF, CHUNK), jnp.int32),
            pltpu.VMEM((NBUF, NGROUP, LANES), jnp.int32),
        ]
        + [pltpu.VMEM((CHUNK, CHUNK), jnp.float32) for _ in range(NBUF)]
        + [pltpu.VMEM((D_MODEL, CHUNK), jnp.float32) for _ in range(NBUF)]
        + [pltpu.SemaphoreType.DMA] * (3 * NBUF),
        compiler_params=_COMPILER_PARAMS,
    )
    def emb_kernel(x_hbm, table_hbm, out_hbm, xraw, gidx, cbv, *rest):
        rows_g = rest[:NBUF]
        rows_s = rest[NBUF : 2 * NBUF]
        sems = rest[2 * NBUF :]
        gsem, ssem, xsem = sems[:NBUF], sems[NBUF : 2 * NBUF], sems[2 * NBUF :]
        wid = lax.axis_index("s") * NUM_CORES + lax.axis_index("c")
        base_block = wid * n_chunks

        iota = lax.iota(jnp.int32, LANES)
        il_vecs = [iota + g * LANES for g in range(NGROUP)]

        def fire_xstage(c, k):
            pltpu.async_copy(x_hbm.at[base_block + c], xraw.at[k], xsem[k])

        def wait_xstage(c, k):
            pltpu.make_async_copy(
                x_hbm.at[base_block + c], xraw.at[k], xsem[k]
            ).wait()

        def prep(k):
            for g in range(NGROUP):
                sl = pl.ds(g * LANES, LANES)
                raw = xraw[k, sl]
                gidx[k, sl] = lax.shift_right_logical(raw, 1)
                cbv[k, g] = lax.shift_left(
                    lax.bitwise_and(raw, jnp.int32(1)), 6
                )

        def fire_gather(c, k):
            pltpu.async_copy(table_hbm.at[gidx.at[k]], rows_g[k], gsem[k])

        def wait_gather(c, k):
            pltpu.make_async_copy(
                table_hbm.at[gidx.at[k]], rows_g[k], gsem[k]
            ).wait()

        def out_slices(c, k):
            b = base_block + c
            j = b // i_blocks
            ihi = b % i_blocks
            return [
                (rows_s[k].at[pl.ds(dh * SUBLANES, SUBLANES)],
                 out_hbm.at[j, dh, ihi])
                for dh in range(d_blocks)
            ]

        def fire_scatter(c, k):
            for src, dst in out_slices(c, k):
                pltpu.async_copy(src, dst, ssem[k])

        def wait_scatter(k):
            pltpu.make_async_copy(
                table_hbm.at[pl.ds(0, D_MODEL)], rows_s[k], ssem[k]
            ).wait()

        def scale(k):
            cols = [cbv[k, g] for g in range(NGROUP)]

            def d_body(d, carry):
                for g in range(NGROUP):
                    v = plsc.load_gather(
                        rows_g[k], [il_vecs[g], cols[g] + d]
                    )
                    rows_s[k][d, pl.ds(g * LANES, LANES)] = v
                return carry

            lax.fori_loop(0, D_MODEL, d_body, 0, unroll=2)

        def round_body(r, carry):
            for k in range(NBUF):
                c = r * NBUF + k
                wait_gather(c, k)

                @pl.when(r > 0)
                def _():
                    wait_scatter(k)

                scale(k)
                fire_scatter(c, k)

                @pl.when(r < n_rounds - 1)
                def _():
                    nc = c + NBUF
                    wait_xstage(nc, k)
                    prep(k)
                    fire_gather(nc, k)

                    @pl.when(r < n_rounds - 2)
                    def _():
                        fire_xstage(nc + NBUF, k)

            return carry

        for k in range(NBUF):
            fire_xstage(k, k)
        for k in range(NBUF):
            wait_xstage(k, k)
            prep(k)
            fire_gather(k, k)
            fire_xstage(k + NBUF, k)
        lax.fori_loop(0, n_rounds, round_body, 0)
        for k in range(NBUF):
            wait_scatter(k)

    return emb_kernel


def kernel(x, table):
    b, s = x.shape
    vocab = table.shape[0]
    table8 = _make_format_kernel(vocab)(table.T)
    x_blocked = x.T.reshape(s * b // CHUNK, CHUNK).astype(jnp.int32)
    out5 = _make_gather_kernel(b, s)(x_blocked, table8)
    out = out5.transpose(2, 4, 0, 1, 3).reshape(b, s, D_MODEL)
    return out


# speedup vs baseline: 1.3229x; 1.3229x over previous
import functools

import jax
import jax.numpy as jnp
from jax import lax
from jax.experimental import pallas as pl
from jax.experimental.pallas import tpu as pltpu
from jax.experimental.pallas import tpu_sc as plsc

D_MODEL = 64
SCALE = 8.0
LANES = 16
NUM_CORES = 2
NUM_SUBCORES = 16
NUM_WORKERS = NUM_CORES * NUM_SUBCORES
CHUNK = 128
NBUF = 4
SUBLANES = 8
NGROUP = CHUNK // LANES
KF_NBUF = 3

_COMPILER_PARAMS = pltpu.CompilerParams(
    use_tc_tiling_on_sc=True,
    needs_layout_passes=False,
    disable_bounds_checks=True,
)


@functools.lru_cache(maxsize=None)
def _make_format_kernel(vocab: int):
    n_tiles = (vocab + CHUNK - 1) // CHUNK
    pairs = vocab // 2
    base_cnt = n_tiles // NUM_WORKERS
    rem = n_tiles % NUM_WORKERS
    max_cnt = base_cnt + (1 if rem else 0)
    n_rounds = (max_cnt + KF_NBUF - 1) // KF_NBUF
    last_tile = n_tiles - 1
    last_rows = (pairs - last_tile * (CHUNK // 2)) if vocab % CHUNK else 0
    mesh = plsc.VectorSubcoreMesh(core_axis_name="c", subcore_axis_name="s")

    @functools.partial(
        pl.kernel,
        out_type=jax.ShapeDtypeStruct((pairs, 2 * D_MODEL), jnp.float32),
        mesh=mesh,
        scratch_types=[pltpu.VMEM((D_MODEL, CHUNK), jnp.float32)
                       for _ in range(KF_NBUF)]
        + [pltpu.VMEM((D_MODEL, CHUNK), jnp.float32)
           for _ in range(KF_NBUF)]
        + [pltpu.SemaphoreType.DMA] * (2 * KF_NBUF),
        compiler_params=_COMPILER_PARAMS,
    )
    def fmt_kernel(tt_hbm, out_hbm, *rest):
        src = rest[:KF_NBUF]
        dst = rest[KF_NBUF : 2 * KF_NBUF]
        rsem = rest[2 * KF_NBUF : 3 * KF_NBUF]
        wsem = rest[3 * KF_NBUF :]
        wid = lax.axis_index("s") * NUM_CORES + lax.axis_index("c")
        start = wid * base_cnt + lax.min(wid, rem)
        count = base_cnt + jnp.where(wid < rem, 1, 0)

        iota = lax.iota(jnp.int32, LANES)
        d_vecs = [iota + jj * LANES for jj in range(D_MODEL // LANES)]

        def fire_read(t, k):
            vh = start + t
            pltpu.async_copy(
                tt_hbm.at[:, pl.ds(vh * CHUNK, CHUNK)], src[k], rsem[k]
            )

        def wait_read(t, k):
            vh = start + t
            pltpu.make_async_copy(
                tt_hbm.at[:, pl.ds(vh * CHUNK, CHUNK)], src[k], rsem[k]
            ).wait()

        def fire_write(t, k):
            vh = start + t
            row0 = vh * (CHUNK // 2)

            @pl.when(vh != last_tile)
            def _():
                pltpu.async_copy(
                    dst[k], out_hbm.at[pl.ds(row0, D_MODEL)], wsem[k]
                )

            if last_rows:
                @pl.when(vh == last_tile)
                def _():
                    pltpu.async_copy(
                        dst[k].at[pl.ds(0, last_rows)],
                        out_hbm.at[pl.ds(row0, last_rows)],
                        wsem[k],
                    )

        def wait_write(t, k):
            vh = start + t

            @pl.when(vh != last_tile)
            def _():
                pltpu.make_async_copy(
                    dst[k], out_hbm.at[pl.ds(0, D_MODEL)], wsem[k]
                ).wait()

            if last_rows:
                @pl.when(vh == last_tile)
                def _():
                    pltpu.make_async_copy(
                        dst[k].at[pl.ds(0, last_rows)],
                        out_hbm.at[pl.ds(0, last_rows)],
                        wsem[k],
                    ).wait()

        def transpose(k):
            def vl_body(vl2, carry):
                vs = []
                for h in range(2):
                    col = jnp.full((LANES,), 2 * vl2 + h, jnp.int32)
                    for jj in range(D_MODEL // LANES):
                        vs.append(
                            plsc.load_gather(src[k], [d_vecs[jj], col])
                        )
                for h in range(2):
                    for jj in range(D_MODEL // LANES):
                        sl = pl.ds(h * D_MODEL + jj * LANES, LANES)
                        dst[k][vl2, sl] = vs[h * 4 + jj] * SCALE
                return carry

            lax.fori_loop(0, CHUNK // 2, vl_body, 0, unroll=2)

        def round_body(r, carry):
            for k in range(KF_NBUF):
                t = r * KF_NBUF + k

                @pl.when(t < count)
                def _():
                    wait_read(t, k)

                    @pl.when(r > 0)
                    def _():
                        wait_write(t - KF_NBUF, k)

                    transpose(k)
                    fire_write(t, k)

                    @pl.when(t + KF_NBUF < count)
                    def _():
                        fire_read(t + KF_NBUF, k)

            return carry

        for k in range(KF_NBUF):
            fire_read(k, k)
        lax.fori_loop(0, n_rounds, round_body, 0)
        for k in range(KF_NBUF):
            last_t = count - 1 - lax.rem(count - 1 - k, jnp.int32(KF_NBUF))
            wait_write(last_t, k)

    return fmt_kernel


@functools.lru_cache(maxsize=None)
def _make_gather_kernel(batch: int, seq: int):
    n_rows = batch * seq
    assert batch % CHUNK == 0
    i_blocks = batch // CHUNK
    n_blocks = n_rows // CHUNK
    assert n_blocks % (NUM_WORKERS * NBUF) == 0
    n_chunks = n_blocks // NUM_WORKERS
    n_rounds = n_chunks // NBUF
    d_blocks = D_MODEL // SUBLANES
    mesh = plsc.VectorSubcoreMesh(core_axis_name="c", subcore_axis_name="s")

    @functools.partial(
        pl.kernel,
        out_type=jax.ShapeDtypeStruct(
            (seq, d_blocks, i_blocks, SUBLANES, CHUNK), jnp.float32
        ),
        mesh=mesh,
        scratch_types=[
            pltpu.VMEM((NBUF, CHUNK), jnp.int32),
            pltpu.VMEM((NBUF, CHUNK), jnp.int32),
            pltpu.VMEM((NBUF, NGROUP, LANES), jnp.int32),
        ]
        + [pltpu.VMEM((CHUNK, CHUNK), jnp.float32) for _ in range(NBUF)]
        + [pltpu.VMEM((D_MODEL, CHUNK), jnp.float32) for _ in range(NBUF)]
        + [pltpu.SemaphoreType.DMA] * (3 * NBUF),
        compiler_params=_COMPILER_PARAMS,
    )
    def emb_kernel(x_hbm, table_hbm, out_hbm, xraw, gidx, cbv, *rest):
        rows_g = rest[:NBUF]
        rows_s = rest[NBUF : 2 * NBUF]
        sems = rest[2 * NBUF :]
        gsem, ssem, xsem = sems[:NBUF], sems[NBUF : 2 * NBUF], sems[2 * NBUF :]
        wid = lax.axis_index("s") * NUM_CORES + lax.axis_index("c")
        base_block = wid * n_chunks

        iota = lax.iota(jnp.int32, LANES)
        il_vecs = [iota + g * LANES for g in range(NGROUP)]

        def fire_xstage(c, k):
            pltpu.async_copy(x_hbm.at[base_block + c], xraw.at[k], xsem[k])

        def wait_xstage(c, k):
            pltpu.make_async_copy(
                x_hbm.at[base_block + c], xraw.at[k], xsem[k]
            ).wait()

        def prep(k):
            for g in range(NGROUP):
                sl = pl.ds(g * LANES, LANES)
                raw = xraw[k, sl]
                gidx[k, sl] = lax.shift_right_logical(raw, 1)
                cbv[k, g] = lax.shift_left(
                    lax.bitwise_and(raw, jnp.int32(1)), 6
                )

        def fire_gather(c, k):
            pltpu.async_copy(table_hbm.at[gidx.at[k]], rows_g[k], gsem[k])

        def wait_gather(c, k):
            pltpu.make_async_copy(
                table_hbm.at[gidx.at[k]], rows_g[k], gsem[k]
            ).wait()

        def out_slices(c, k):
            b = base_block + c
            j = b // i_blocks
            ihi = b % i_blocks
            return [
                (rows_s[k].at[pl.ds(dh * SUBLANES, SUBLANES)],
                 out_hbm.at[j, dh, ihi])
                for dh in range(d_blocks)
            ]

        def fire_scatter(c, k):
            for src, dst in out_slices(c, k):
                pltpu.async_copy(src, dst, ssem[k])

        def wait_scatter(k):
            pltpu.make_async_copy(
                table_hbm.at[pl.ds(0, D_MODEL)], rows_s[k], ssem[k]
            ).wait()

        def scale(k):
            cols = [cbv[k, g] for g in range(NGROUP)]

            def d_body(d, carry):
                vs = [
                    plsc.load_gather(rows_g[k], [il_vecs[g], cols[g] + d])
                    for g in range(NGROUP)
                ]
                for g in range(NGROUP):
                    rows_s[k][d, pl.ds(g * LANES, LANES)] = vs[g]
                return carry

            lax.fori_loop(0, D_MODEL, d_body, 0, unroll=2)

        def round_body(r, carry):
            for k in range(NBUF):
                c = r * NBUF + k
                wait_gather(c, k)

                @pl.when(r > 0)
                def _():
                    wait_scatter(k)

                scale(k)
                fire_scatter(c, k)

                @pl.when(r < n_rounds - 1)
                def _():
                    nc = c + NBUF
                    wait_xstage(nc, k)
                    prep(k)
                    fire_gather(nc, k)

                    @pl.when(r < n_rounds - 2)
                    def _():
                        fire_xstage(nc + NBUF, k)

            return carry

        for k in range(NBUF):
            fire_xstage(k, k)
        for k in range(NBUF):
            wait_xstage(k, k)
            prep(k)
            fire_gather(k, k)
            fire_xstage(k + NBUF, k)
        lax.fori_loop(0, n_rounds, round_body, 0)
        for k in range(NBUF):
            wait_scatter(k)

    return emb_kernel


def kernel(x, table):
    b, s = x.shape
    vocab = table.shape[0]
    table8 = _make_format_kernel(vocab)(table.T)
    x_blocked = x.T.reshape(s * b // CHUNK, CHUNK).astype(jnp.int32)
    out5 = _make_gather_kernel(b, s)(x_blocked, table8)
    out = out5.transpose(2, 4, 0, 1, 3).reshape(b, s, D_MODEL)
    return out


# speedup vs baseline: 2.6119x; 1.9744x over previous
import functools

import jax
import jax.numpy as jnp
from jax import lax
from jax.experimental import pallas as pl
from jax.experimental.pallas import tpu as pltpu
from jax.experimental.pallas import tpu_sc as plsc

D_MODEL = 64
SCALE = 8.0
LANES = 16
NUM_CORES = 2
NUM_SUBCORES = 16
NUM_WORKERS = NUM_CORES * NUM_SUBCORES
CHUNK = 128
NBUF = 4
SUBLANES = 8
NGROUP = CHUNK // LANES
KF_NBUF = 3

_COMPILER_PARAMS = pltpu.CompilerParams(
    use_tc_tiling_on_sc=True,
    needs_layout_passes=False,
    disable_bounds_checks=True,
)


@functools.lru_cache(maxsize=None)
def _make_format_kernel(vocab: int):
    n_tiles = (vocab + CHUNK - 1) // CHUNK
    pairs = vocab // 2
    base_cnt = n_tiles // NUM_WORKERS
    rem = n_tiles % NUM_WORKERS
    max_cnt = base_cnt + (1 if rem else 0)
    n_rounds = (max_cnt + KF_NBUF - 1) // KF_NBUF
    last_tile = n_tiles - 1
    last_rows = (pairs - last_tile * (CHUNK // 2)) if vocab % CHUNK else 0
    mesh = plsc.VectorSubcoreMesh(core_axis_name="c", subcore_axis_name="s")

    @functools.partial(
        pl.kernel,
        out_type=jax.ShapeDtypeStruct((pairs, 2 * D_MODEL), jnp.float32),
        mesh=mesh,
        scratch_types=[pltpu.VMEM((D_MODEL, CHUNK), jnp.float32)
                       for _ in range(KF_NBUF)]
        + [pltpu.VMEM((D_MODEL, CHUNK), jnp.float32)
           for _ in range(KF_NBUF)]
        + [pltpu.SemaphoreType.DMA] * (2 * KF_NBUF),
        compiler_params=_COMPILER_PARAMS,
    )
    def fmt_kernel(tt_hbm, out_hbm, *rest):
        src = rest[:KF_NBUF]
        dst = rest[KF_NBUF : 2 * KF_NBUF]
        rsem = rest[2 * KF_NBUF : 3 * KF_NBUF]
        wsem = rest[3 * KF_NBUF :]
        wid = lax.axis_index("s") * NUM_CORES + lax.axis_index("c")
        start = wid * base_cnt + lax.min(wid, rem)
        count = base_cnt + jnp.where(wid < rem, 1, 0)

        iota = lax.iota(jnp.int32, LANES)
        d_vecs = [iota + jj * LANES for jj in range(D_MODEL // LANES)]

        def fire_read(t, k):
            vh = start + t
            pltpu.async_copy(
                tt_hbm.at[:, pl.ds(vh * CHUNK, CHUNK)], src[k], rsem[k]
            )

        def wait_read(t, k):
            vh = start + t
            pltpu.make_async_copy(
                tt_hbm.at[:, pl.ds(vh * CHUNK, CHUNK)], src[k], rsem[k]
            ).wait()

        def fire_write(t, k):
            vh = start + t
            row0 = vh * (CHUNK // 2)

            @pl.when(vh != last_tile)
            def _():
                pltpu.async_copy(
                    dst[k], out_hbm.at[pl.ds(row0, D_MODEL)], wsem[k]
                )

            if last_rows:
                @pl.when(vh == last_tile)
                def _():
                    pltpu.async_copy(
                        dst[k].at[pl.ds(0, last_rows)],
                        out_hbm.at[pl.ds(row0, last_rows)],
                        wsem[k],
                    )

        def wait_write(t, k):
            vh = start + t

            @pl.when(vh != last_tile)
            def _():
                pltpu.make_async_copy(
                    dst[k], out_hbm.at[pl.ds(0, D_MODEL)], wsem[k]
                ).wait()

            if last_rows:
                @pl.when(vh == last_tile)
                def _():
                    pltpu.make_async_copy(
                        dst[k].at[pl.ds(0, last_rows)],
                        out_hbm.at[pl.ds(0, last_rows)],
                        wsem[k],
                    ).wait()

        def transpose(k):
            def dd_body(dd, carry):
                m = lax.bitwise_and(iota + dd, jnp.int32(LANES - 1))
                mrow = lax.shift_right_logical(m, 1)
                mcol = lax.shift_left(lax.bitwise_and(m, 1), 6)
                for dq in range(D_MODEL // LANES):
                    ld_row = iota + dq * LANES
                    for vq in range(CHUNK // LANES):
                        ld_col = m + vq * LANES
                        v = plsc.load_gather(src[k], [ld_row, ld_col])
                        st_row = mrow + (vq * LANES) // 2
                        st_col = mcol + (iota + dq * LANES)
                        plsc.store_scatter(
                            dst[k], [st_row, st_col], v * SCALE
                        )
                return carry

            lax.fori_loop(0, LANES, dd_body, 0)

        def round_body(r, carry):
            for k in range(KF_NBUF):
                t = r * KF_NBUF + k

                @pl.when(t < count)
                def _():
                    wait_read(t, k)

                    @pl.when(r > 0)
                    def _():
                        wait_write(t - KF_NBUF, k)

                    transpose(k)
                    fire_write(t, k)

                    @pl.when(t + KF_NBUF < count)
                    def _():
                        fire_read(t + KF_NBUF, k)

            return carry

        for k in range(KF_NBUF):
            fire_read(k, k)
        lax.fori_loop(0, n_rounds, round_body, 0)
        for k in range(KF_NBUF):
            last_t = count - 1 - lax.rem(count - 1 - k, jnp.int32(KF_NBUF))
            wait_write(last_t, k)

    return fmt_kernel


@functools.lru_cache(maxsize=None)
def _make_gather_kernel(batch: int, seq: int):
    n_rows = batch * seq
    assert batch % CHUNK == 0
    i_blocks = batch // CHUNK
    n_blocks = n_rows // CHUNK
    assert n_blocks % (NUM_WORKERS * NBUF) == 0
    n_chunks = n_blocks // NUM_WORKERS
    n_rounds = n_chunks // NBUF
    d_blocks = D_MODEL // SUBLANES
    mesh = plsc.VectorSubcoreMesh(core_axis_name="c", subcore_axis_name="s")

    @functools.partial(
        pl.kernel,
        out_type=jax.ShapeDtypeStruct(
            (seq, d_blocks, i_blocks, SUBLANES, CHUNK), jnp.float32
        ),
        mesh=mesh,
        scratch_types=[
            pltpu.VMEM((NBUF, CHUNK), jnp.int32),
            pltpu.VMEM((NBUF, CHUNK), jnp.int32),
            pltpu.VMEM((NBUF, NGROUP, LANES), jnp.int32),
        ]
        + [pltpu.VMEM((CHUNK, CHUNK), jnp.float32) for _ in range(NBUF)]
        + [pltpu.VMEM((D_MODEL, CHUNK), jnp.float32) for _ in range(NBUF)]
        + [pltpu.SemaphoreType.DMA] * (3 * NBUF),
        compiler_params=_COMPILER_PARAMS,
    )
    def emb_kernel(x_hbm, table_hbm, out_hbm, xraw, gidx, cbv, *rest):
        rows_g = rest[:NBUF]
        rows_s = rest[NBUF : 2 * NBUF]
        sems = rest[2 * NBUF :]
        gsem, ssem, xsem = sems[:NBUF], sems[NBUF : 2 * NBUF], sems[2 * NBUF :]
        wid = lax.axis_index("s") * NUM_CORES + lax.axis_index("c")
        base_block = wid * n_chunks

        iota = lax.iota(jnp.int32, LANES)
        il_vecs = [iota + g * LANES for g in range(NGROUP)]

        def fire_xstage(c, k):
            pltpu.async_copy(x_hbm.at[base_block + c], xraw.at[k], xsem[k])

        def wait_xstage(c, k):
            pltpu.make_async_copy(
                x_hbm.at[base_block + c], xraw.at[k], xsem[k]
            ).wait()

        def prep(k):
            for g in range(NGROUP):
                sl = pl.ds(g * LANES, LANES)
                raw = xraw[k, sl]
                gidx[k, sl] = lax.shift_right_logical(raw, 1)
                cbv[k, g] = lax.shift_left(
                    lax.bitwise_and(raw, jnp.int32(1)), 6
                )

        def fire_gather(c, k):
            pltpu.async_copy(table_hbm.at[gidx.at[k]], rows_g[k], gsem[k])

        def wait_gather(c, k):
            pltpu.make_async_copy(
                table_hbm.at[gidx.at[k]], rows_g[k], gsem[k]
            ).wait()

        def out_slices(c, k):
            b = base_block + c
            j = b // i_blocks
            ihi = b % i_blocks
            return [
                (rows_s[k].at[pl.ds(dh * SUBLANES, SUBLANES)],
                 out_hbm.at[j, dh, ihi])
                for dh in range(d_blocks)
            ]

        def fire_scatter(c, k):
            for src, dst in out_slices(c, k):
                pltpu.async_copy(src, dst, ssem[k])

        def wait_scatter(k):
            pltpu.make_async_copy(
                table_hbm.at[pl.ds(0, D_MODEL)], rows_s[k], ssem[k]
            ).wait()

        def scale(k):
            cols = [cbv[k, g] for g in range(NGROUP)]

            def dd_body(dd, carry):
                m = lax.bitwise_and(iota + dd, jnp.int32(LANES - 1))
                for dq in range(D_MODEL // LANES):
                    md = m + dq * LANES
                    for g in range(NGROUP):
                        v = plsc.load_gather(
                            rows_g[k], [il_vecs[g], cols[g] + md]
                        )
                        plsc.store_scatter(
                            rows_s[k], [md, il_vecs[g]], v
                        )
                return carry

            lax.fori_loop(0, LANES, dd_body, 0)

        def round_body(r, carry):
            for k in range(NBUF):
                c = r * NBUF + k
                wait_gather(c, k)

                @pl.when(r > 0)
                def _():
                    wait_scatter(k)

                scale(k)
                fire_scatter(c, k)

                @pl.when(r < n_rounds - 1)
                def _():
                    nc = c + NBUF
                    wait_xstage(nc, k)
                    prep(k)
                    fire_gather(nc, k)

                    @pl.when(r < n_rounds - 2)
                    def _():
                        fire_xstage(nc + NBUF, k)

            return carry

        for k in range(NBUF):
            fire_xstage(k, k)
        for k in range(NBUF):
            wait_xstage(k, k)
            prep(k)
            fire_gather(k, k)
            fire_xstage(k + NBUF, k)
        lax.fori_loop(0, n_rounds, round_body, 0)
        for k in range(NBUF):
            wait_scatter(k)

    return emb_kernel


def kernel(x, table):
    b, s = x.shape
    vocab = table.shape[0]
    table8 = _make_format_kernel(vocab)(table.T)
    x_blocked = x.T.reshape(s * b // CHUNK, CHUNK).astype(jnp.int32)
    out5 = _make_gather_kernel(b, s)(x_blocked, table8)
    out = out5.transpose(2, 4, 0, 1, 3).reshape(b, s, D_MODEL)
    return out
